# baseline (speedup 1.0000x reference)
import functools

import jax
import jax.numpy as jnp
from jax import lax
from jax.experimental import pallas as pl
from jax.experimental.pallas import tpu as pltpu
from jax.experimental.pallas import tpu_sc as plsc

VOCAB = 1000000
EMB_DIM = 64
BATCH = 16384

NC = 2
NS = 16
NW = NC * NS
B_PER_W = BATCH // NW

KS = 10
KD = 6
NGROUP = B_PER_W // (KS + KD)
NSROWS = NGROUP * KS

_mesh = plsc.VectorSubcoreMesh(core_axis_name="c", subcore_axis_name="s")


@functools.partial(
    pl.kernel,
    mesh=_mesh,
    out_type=jax.ShapeDtypeStruct((BATCH, EMB_DIM), jnp.float32),
    scratch_types=[
        pltpu.VMEM_SHARED((BATCH,), jnp.int32),
        pltpu.SMEM((B_PER_W,), jnp.int32),
        pltpu.VMEM((NSROWS, EMB_DIM), jnp.float32),
        pltpu.SemaphoreType.DMA,
        pltpu.SemaphoreType.DMA,
    ],
)
def _gather_rows(table_hbm, idx_hbm, out_hbm, idx_sp, idx_s, rows_v, sem_s,
                 sem_d):
    cid = lax.axis_index("c")
    sid = lax.axis_index("s")
    wid = sid * NC + cid
    base = wid * B_PER_W

    @pl.when(sid == 0)
    def _():
        pltpu.sync_copy(idx_hbm, idx_sp)

    plsc.subcore_barrier()
    pltpu.sync_copy(idx_sp.at[pl.ds(base, B_PER_W)], idx_s)

    def group(g, carry):
        s0 = g * KS
        d0 = NSROWS + g * KD
        copies = [
            pltpu.async_copy(
                table_hbm.at[idx_s[s0 + j]], rows_v.at[s0 + j], sem_s
            )
            for j in range(KS)
        ] + [
            pltpu.async_copy(
                table_hbm.at[idx_s[d0 + j]], out_hbm.at[base + d0 + j], sem_d
            )
            for j in range(KD)
        ]
        for cp in copies:
            cp.wait()
        return carry

    lax.fori_loop(0, NGROUP, group, 0)
    pltpu.sync_copy(rows_v, out_hbm.at[pl.ds(base, NSROWS)])


def kernel(indices, table):
    return _gather_rows(table, indices.astype(jnp.int32))
